# baseline (speedup 1.0000x reference)
import functools

import jax
import jax.numpy as jnp
from jax import lax
from jax.experimental import pallas as pl
from jax.experimental.pallas import tpu as pltpu
from jax.experimental.pallas import tpu_sc as plsc

N = 10000
E = 320000
NC = 2
NS = 16
NW = NC * NS
CHUNK = 128
EC = 80
KB = 8
E_PAD = NW * EC * CHUNK
N_PAD = 10240
RPT = N_PAD // NS
DEG_W = 8

_mesh = plsc.VectorSubcoreMesh(
    core_axis_name="c", subcore_axis_name="s", num_cores=NC, num_subcores=NS
)
_sc_params = pltpu.CompilerParams(use_tc_tiling_on_sc=False)


def _make_agg(D):
  @functools.partial(
      pl.kernel,
      out_type=jax.ShapeDtypeStruct((NC, N_PAD, D), jnp.float32),
      mesh=_mesh,
      scratch_types=[
          pltpu.VMEM((EC, CHUNK), jnp.int32),
          pltpu.VMEM((EC, CHUNK), jnp.int32),
          pltpu.VMEM((KB, CHUNK, D), jnp.float32),
          pltpu.VMEM_SHARED((N_PAD, D), jnp.float32),
          pltpu.SemaphoreType.DMA((KB,)),
          pltpu.SemaphoreType.DMA((KB,)),
      ],
      compiler_params=_sc_params,
  )
  def agg(xs_hbm, src_hbm, dst_hbm, zeros_hbm, out_hbm,
          src_v, dst_v, rows_v, acc_sh, gsem, ssem):
    cid = lax.axis_index("c")
    sid = lax.axis_index("s")
    wid = sid * NC + cid
    pltpu.sync_copy(src_hbm.at[wid], src_v)
    pltpu.sync_copy(dst_hbm.at[wid], dst_v)
    pltpu.sync_copy(zeros_hbm, acc_sh.at[pl.ds(sid * RPT, RPT)])
    plsc.subcore_barrier()

    def round_body(r, carry):
      base = r * KB
      gd = [pltpu.async_copy(xs_hbm.at[src_v.at[base + b]], rows_v.at[b],
                             gsem.at[b]) for b in range(KB)]
      sd = []
      for b in range(KB):
        gd[b].wait()
        sd.append(pltpu.async_copy(rows_v.at[b], acc_sh.at[dst_v.at[base + b]],
                                   ssem.at[b], add=True))
      for b in range(KB):
        sd[b].wait()
      return carry

    lax.fori_loop(0, EC // KB, round_body, 0)
    plsc.subcore_barrier()
    pltpu.sync_copy(acc_sh.at[pl.ds(sid * RPT, RPT)],
                    out_hbm.at[cid, pl.ds(sid * RPT, RPT)])

  return agg


@functools.partial(
    pl.kernel,
    out_type=jax.ShapeDtypeStruct((NC, N_PAD, DEG_W), jnp.float32),
    mesh=_mesh,
    scratch_types=[
        pltpu.VMEM((EC, CHUNK), jnp.int32),
        pltpu.VMEM((CHUNK, DEG_W), jnp.float32),
        pltpu.VMEM_SHARED((N_PAD, DEG_W), jnp.float32),
    ],
    compiler_params=_sc_params,
)
def _deg_kernel(dst_hbm, ones_hbm, zeros_hbm, out_hbm,
                dst_v, ones_v, acc_sh):
  cid = lax.axis_index("c")
  sid = lax.axis_index("s")
  wid = sid * NC + cid
  pltpu.sync_copy(dst_hbm.at[wid], dst_v)
  pltpu.sync_copy(ones_hbm, ones_v)
  pltpu.sync_copy(zeros_hbm, acc_sh.at[pl.ds(sid * RPT, RPT)])
  plsc.subcore_barrier()

  def body(j, carry):
    pltpu.sync_copy(ones_v, acc_sh.at[dst_v.at[j]], add=True)
    return carry

  lax.fori_loop(0, EC, body, 0)
  plsc.subcore_barrier()
  pltpu.sync_copy(acc_sh.at[pl.ds(sid * RPT, RPT)],
                  out_hbm.at[cid, pl.ds(sid * RPT, RPT)])


def _tc_first(degA, degB, x, W):
  def body(da, db, x_ref, w_ref, xs_out, isd_out):
    deg = jnp.maximum(da[:, 0:1] + db[:, 0:1], 1.0)
    isd = lax.rsqrt(deg)
    t = jnp.dot(x_ref[...], w_ref[...], preferred_element_type=jnp.float32)
    xs_out[...] = t * isd
    isd_out[...] = jnp.broadcast_to(isd, (N, DEG_W))
  return pl.pallas_call(
      body,
      out_shape=[jax.ShapeDtypeStruct((N, W.shape[1]), jnp.float32),
                 jax.ShapeDtypeStruct((N, DEG_W), jnp.float32)],
  )(degA, degB, x, W)


def _tc_mid(aggA, aggB, isd, b, W, want_h):
  def body(aa, ab, isd_ref, b_ref, w_ref, *outs):
    isd = isd_ref[:, 0:1]
    h = jnp.maximum(isd * (aa[...] + ab[...]) + b_ref[...], 0.0)
    t = jnp.dot(h, w_ref[...], preferred_element_type=jnp.float32)
    outs[0][...] = t * isd
    if want_h:
      outs[1][...] = h
  out_shape = [jax.ShapeDtypeStruct((N, W.shape[1]), jnp.float32)]
  if want_h:
    out_shape.append(jax.ShapeDtypeStruct((N, b.shape[0]), jnp.float32))
  return pl.pallas_call(body, out_shape=out_shape)(
      aggA, aggB, isd, b.reshape(1, -1), W)


def _tc_scale(aggA, aggB, isd, b):
  def body(aa, ab, isd_ref, b_ref, out):
    isd = isd_ref[:, 0:1]
    out[...] = jnp.maximum(isd * (aa[...] + ab[...]) + b_ref[...], 0.0) * isd
  return pl.pallas_call(
      body, out_shape=jax.ShapeDtypeStruct((N, aggA.shape[1]), jnp.float32)
  )(aggA, aggB, isd, b.reshape(1, -1))


def _tc_last(aggA, aggB, isd, W, b):
  def body(aa, ab, isd_ref, w_ref, b_ref, out):
    isd = isd_ref[:, 0:1]
    t = jnp.dot(isd * (aa[...] + ab[...]), w_ref[...],
                preferred_element_type=jnp.float32)
    out[...] = t + b_ref[...]
  return pl.pallas_call(
      body, out_shape=jax.ShapeDtypeStruct((N, W.shape[1]), jnp.float32)
  )(aggA, aggB, isd, W, b.reshape(1, -1))


_agg64 = _make_agg(64)
_agg32 = _make_agg(32)


def _pad_rows(x):
  return jnp.pad(x, ((0, N_PAD - N), (0, 0)))


def kernel(edge_index, edge_weight, We1, be1, We2, be2, Wd1, bd1, Wd2, bd2):
  src = edge_index[0].astype(jnp.int32)
  dst = edge_index[1].astype(jnp.int32)
  fill = jnp.full((E_PAD - E,), N, jnp.int32)
  src_p = jnp.concatenate([src, fill]).reshape(NW, EC, CHUNK)
  dst_p = jnp.concatenate([dst, fill]).reshape(NW, EC, CHUNK)
  ones_rows = jnp.ones((CHUNK, DEG_W), jnp.float32)
  zeros8 = jnp.zeros((RPT, DEG_W), jnp.float32)
  zeros32 = jnp.zeros((RPT, 32), jnp.float32)
  zeros64 = jnp.zeros((RPT, 64), jnp.float32)

  deg2 = _deg_kernel(dst_p, ones_rows, zeros8)
  xs1, isd = _tc_first(deg2[0, :N], deg2[1, :N], edge_weight, We1)

  agg1 = _agg64(_pad_rows(xs1), src_p, dst_p, zeros64)
  xs2 = _tc_mid(agg1[0, :N], agg1[1, :N], isd, be1, We2, want_h=False)[0]

  agg2 = _agg32(_pad_rows(xs2), src_p, dst_p, zeros32)
  xs3, z = _tc_mid(agg2[0, :N], agg2[1, :N], isd, be2, Wd1, want_h=True)

  agg3 = _agg64(_pad_rows(xs3), src_p, dst_p, zeros64)
  xs4 = _tc_scale(agg3[0, :N], agg3[1, :N], isd, bd1)

  agg4 = _agg64(_pad_rows(xs4), src_p, dst_p, zeros64)
  recon = _tc_last(agg4[0, :N], agg4[1, :N], isd, Wd2, bd2)

  return (recon, z)


# speedup vs baseline: 1.8829x; 1.8829x over previous
import functools

import jax
import jax.numpy as jnp
from jax import lax
from jax.experimental import pallas as pl
from jax.experimental.pallas import tpu as pltpu
from jax.experimental.pallas import tpu_sc as plsc

N = 10000
E = 320000
NC = 2
NS = 16
NW = NC * NS
CHUNK = 128
EC = 80
E_PAD = NW * EC * CHUNK
N_PAD = 10240
RPT = N_PAD // NS
DEG_W = 8

_mesh = plsc.VectorSubcoreMesh(
    core_axis_name="c", subcore_axis_name="s", num_cores=NC, num_subcores=NS
)
_sc_params = pltpu.CompilerParams(use_tc_tiling_on_sc=False)


def _make_agg(D):
  KB = 2 if D == 64 else 4

  @functools.partial(
      pl.kernel,
      out_type=jax.ShapeDtypeStruct((NC, N_PAD, D), jnp.float32),
      mesh=_mesh,
      scratch_types=[
          pltpu.VMEM((EC, CHUNK), jnp.int32),
          pltpu.VMEM((EC, CHUNK), jnp.int32),
          pltpu.VMEM((KB, CHUNK, D), jnp.float32),
          pltpu.VMEM_SHARED((N_PAD, D), jnp.float32),
          pltpu.VMEM_SHARED((N_PAD, D), jnp.float32),
          pltpu.SemaphoreType.DMA((KB,)),
          pltpu.SemaphoreType.DMA((KB,)),
      ],
      compiler_params=_sc_params,
  )
  def agg(xs_hbm, src_hbm, dst_hbm, zeros_hbm, out_hbm,
          src_v, dst_v, rows_v, acc_sh, table_sh, gsem, ssem):
    cid = lax.axis_index("c")
    sid = lax.axis_index("s")
    wid = sid * NC + cid
    pltpu.sync_copy(src_hbm.at[wid], src_v)
    pltpu.sync_copy(dst_hbm.at[wid], dst_v)
    pltpu.sync_copy(xs_hbm.at[pl.ds(sid * RPT, RPT)],
                    table_sh.at[pl.ds(sid * RPT, RPT)])
    pltpu.sync_copy(zeros_hbm, acc_sh.at[pl.ds(sid * RPT, RPT)])
    plsc.subcore_barrier()

    def round_body(r, carry):
      base = r * KB
      gd = [pltpu.async_copy(table_sh.at[src_v.at[base + b]], rows_v.at[b],
                             gsem.at[b]) for b in range(KB)]
      sd = []
      for b in range(KB):
        gd[b].wait()
        sd.append(pltpu.async_copy(rows_v.at[b], acc_sh.at[dst_v.at[base + b]],
                                   ssem.at[b], add=True))
      for b in range(KB):
        sd[b].wait()
      return carry

    lax.fori_loop(0, EC // KB, round_body, 0)
    plsc.subcore_barrier()
    pltpu.sync_copy(acc_sh.at[pl.ds(sid * RPT, RPT)],
                    out_hbm.at[cid, pl.ds(sid * RPT, RPT)])

  return agg


@functools.partial(
    pl.kernel,
    out_type=jax.ShapeDtypeStruct((NC, N_PAD, DEG_W), jnp.float32),
    mesh=_mesh,
    scratch_types=[
        pltpu.VMEM((EC, CHUNK), jnp.int32),
        pltpu.VMEM((CHUNK, DEG_W), jnp.float32),
        pltpu.VMEM_SHARED((N_PAD, DEG_W), jnp.float32),
    ],
    compiler_params=_sc_params,
)
def _deg_kernel(dst_hbm, ones_hbm, zeros_hbm, out_hbm,
                dst_v, ones_v, acc_sh):
  cid = lax.axis_index("c")
  sid = lax.axis_index("s")
  wid = sid * NC + cid
  pltpu.sync_copy(dst_hbm.at[wid], dst_v)
  pltpu.sync_copy(ones_hbm, ones_v)
  pltpu.sync_copy(zeros_hbm, acc_sh.at[pl.ds(sid * RPT, RPT)])
  plsc.subcore_barrier()

  def body(j, carry):
    pltpu.sync_copy(ones_v, acc_sh.at[dst_v.at[j]], add=True)
    return carry

  lax.fori_loop(0, EC, body, 0)
  plsc.subcore_barrier()
  pltpu.sync_copy(acc_sh.at[pl.ds(sid * RPT, RPT)],
                  out_hbm.at[cid, pl.ds(sid * RPT, RPT)])


def _tc_first(degA, degB, x, W):
  def body(da, db, x_ref, w_ref, xs_out, isd_out):
    deg = jnp.maximum(da[:, 0:1] + db[:, 0:1], 1.0)
    isd = lax.rsqrt(deg)
    t = jnp.dot(x_ref[...], w_ref[...], preferred_element_type=jnp.float32)
    xs_out[...] = t * isd
    isd_out[...] = jnp.broadcast_to(isd, (N, DEG_W))
  return pl.pallas_call(
      body,
      out_shape=[jax.ShapeDtypeStruct((N, W.shape[1]), jnp.float32),
                 jax.ShapeDtypeStruct((N, DEG_W), jnp.float32)],
  )(degA, degB, x, W)


def _tc_mid(aggA, aggB, isd, b, W, want_h):
  def body(aa, ab, isd_ref, b_ref, w_ref, *outs):
    isd = isd_ref[:, 0:1]
    h = jnp.maximum(isd * (aa[...] + ab[...]) + b_ref[...], 0.0)
    t = jnp.dot(h, w_ref[...], preferred_element_type=jnp.float32)
    outs[0][...] = t * isd
    if want_h:
      outs[1][...] = h
  out_shape = [jax.ShapeDtypeStruct((N, W.shape[1]), jnp.float32)]
  if want_h:
    out_shape.append(jax.ShapeDtypeStruct((N, b.shape[0]), jnp.float32))
  return pl.pallas_call(body, out_shape=out_shape)(
      aggA, aggB, isd, b.reshape(1, -1), W)


def _tc_scale(aggA, aggB, isd, b):
  def body(aa, ab, isd_ref, b_ref, out):
    isd = isd_ref[:, 0:1]
    out[...] = jnp.maximum(isd * (aa[...] + ab[...]) + b_ref[...], 0.0) * isd
  return pl.pallas_call(
      body, out_shape=jax.ShapeDtypeStruct((N, aggA.shape[1]), jnp.float32)
  )(aggA, aggB, isd, b.reshape(1, -1))


def _tc_last(aggA, aggB, isd, W, b):
  def body(aa, ab, isd_ref, w_ref, b_ref, out):
    isd = isd_ref[:, 0:1]
    t = jnp.dot(isd * (aa[...] + ab[...]), w_ref[...],
                preferred_element_type=jnp.float32)
    out[...] = t + b_ref[...]
  return pl.pallas_call(
      body, out_shape=jax.ShapeDtypeStruct((N, W.shape[1]), jnp.float32)
  )(aggA, aggB, isd, W, b.reshape(1, -1))


_agg64 = _make_agg(64)
_agg32 = _make_agg(32)


def _pad_rows(x):
  return jnp.pad(x, ((0, N_PAD - N), (0, 0)))


def kernel(edge_index, edge_weight, We1, be1, We2, be2, Wd1, bd1, Wd2, bd2):
  src = edge_index[0].astype(jnp.int32)
  dst = edge_index[1].astype(jnp.int32)
  fill = jnp.full((E_PAD - E,), N, jnp.int32)
  src_p = jnp.concatenate([src, fill]).reshape(NW, EC, CHUNK)
  dst_p = jnp.concatenate([dst, fill]).reshape(NW, EC, CHUNK)
  ones_rows = jnp.ones((CHUNK, DEG_W), jnp.float32)
  zeros8 = jnp.zeros((RPT, DEG_W), jnp.float32)
  zeros32 = jnp.zeros((RPT, 32), jnp.float32)
  zeros64 = jnp.zeros((RPT, 64), jnp.float32)

  deg2 = _deg_kernel(dst_p, ones_rows, zeros8)
  xs1, isd = _tc_first(deg2[0, :N], deg2[1, :N], edge_weight, We1)

  agg1 = _agg64(_pad_rows(xs1), src_p, dst_p, zeros64)
  xs2 = _tc_mid(agg1[0, :N], agg1[1, :N], isd, be1, We2, want_h=False)[0]

  agg2 = _agg32(_pad_rows(xs2), src_p, dst_p, zeros32)
  xs3, z = _tc_mid(agg2[0, :N], agg2[1, :N], isd, be2, Wd1, want_h=True)

  agg3 = _agg64(_pad_rows(xs3), src_p, dst_p, zeros64)
  xs4 = _tc_scale(agg3[0, :N], agg3[1, :N], isd, bd1)

  agg4 = _agg64(_pad_rows(xs4), src_p, dst_p, zeros64)
  recon = _tc_last(agg4[0, :N], agg4[1, :N], isd, Wd2, bd2)

  return (recon, z)


# speedup vs baseline: 2.5226x; 1.3397x over previous
import functools

import jax
import jax.numpy as jnp
from jax import lax
from jax.experimental import pallas as pl
from jax.experimental.pallas import tpu as pltpu
from jax.experimental.pallas import tpu_sc as plsc

N = 10000
E = 320000
NC = 2
NS = 16
NW = NC * NS
CHUNK = 128
EC = 81
KB = 3
E_PAD = NW * EC * CHUNK
N_PAD = 10240
RPT = N_PAD // NS
DEG_W = 8

_mesh = plsc.VectorSubcoreMesh(
    core_axis_name="c", subcore_axis_name="s", num_cores=NC, num_subcores=NS
)
_sc_params = pltpu.CompilerParams(use_tc_tiling_on_sc=False)


def _make_agg(D):
  R = EC // KB

  @functools.partial(
      pl.kernel,
      out_type=jax.ShapeDtypeStruct((NC, N_PAD, D), jnp.float32),
      mesh=_mesh,
      scratch_types=[
          pltpu.VMEM((EC, CHUNK), jnp.int32),
          pltpu.VMEM((EC, CHUNK), jnp.int32),
          pltpu.VMEM((KB * CHUNK, D), jnp.float32),
          pltpu.VMEM_SHARED((N_PAD, D), jnp.float32),
          pltpu.VMEM_SHARED((N_PAD, D), jnp.float32),
          pltpu.SemaphoreType.DMA((KB,)),
          pltpu.SemaphoreType.DMA((KB,)),
      ],
      compiler_params=_sc_params,
  )
  def agg(xs_hbm, src_hbm, dst_hbm, out_hbm,
          src_v, dst_v, rows_v, acc_sh, table_sh, gsem, ssem):
    cid = lax.axis_index("c")
    sid = lax.axis_index("s")
    wid = sid * NC + cid

    def rbuf(b):
      return rows_v.at[pl.ds(b * CHUNK, CHUNK)]

    def g_start(j, b):
      pltpu.async_copy(table_sh.at[src_v.at[j]], rbuf(b), gsem.at[b])

    def g_wait(j, b):
      pltpu.make_async_copy(table_sh.at[src_v.at[j]], rbuf(b),
                            gsem.at[b]).wait()

    def s_start(j, b):
      pltpu.async_copy(rbuf(b), acc_sh.at[dst_v.at[j]], ssem.at[b], add=True)

    def s_wait(j, b):
      pltpu.make_async_copy(rbuf(b), acc_sh.at[dst_v.at[j]],
                            ssem.at[b]).wait()

    zv = jnp.zeros((16,), jnp.float32)

    def zrow(i, carry):
      for k in range(D // 16):
        rows_v[i, pl.ds(k * 16, 16)] = zv
      return carry

    lax.fori_loop(0, KB * CHUNK, zrow, 0)
    stage = [
        pltpu.async_copy(src_hbm.at[wid], src_v, gsem.at[0]),
        pltpu.async_copy(dst_hbm.at[wid], dst_v, gsem.at[1]),
        pltpu.async_copy(xs_hbm.at[pl.ds(sid * RPT, RPT)],
                         table_sh.at[pl.ds(sid * RPT, RPT)], gsem.at[2]),
    ]
    off = 0
    zi = 0
    while off < RPT:
      n = min(KB * CHUNK, RPT - off)
      stage.append(
          pltpu.async_copy(rows_v.at[pl.ds(0, n)],
                           acc_sh.at[pl.ds(sid * RPT + off, n)], ssem.at[zi]))
      zi += 1
      off += n
    for d in stage:
      d.wait()
    plsc.subcore_barrier()

    g_start(0, 0)

    def round_body(r, carry):
      for b in range(KB):
        j = r * KB + b
        bn = (b + 1) % KB
        if b == KB - 1:
          @pl.when(r < R - 1)
          def _():
            s_wait(j + 1 - KB, bn)
            g_start(j + 1, bn)
        else:
          @pl.when(r > 0)
          def _():
            s_wait(j + 1 - KB, bn)
          g_start(j + 1, bn)
        g_wait(j, b)
        s_start(j, b)
      return carry

    lax.fori_loop(0, R, round_body, 0)
    for b in range(KB):
      s_wait(EC - KB + b, b)
    plsc.subcore_barrier()
    pltpu.sync_copy(acc_sh.at[pl.ds(sid * RPT, RPT)],
                    out_hbm.at[cid, pl.ds(sid * RPT, RPT)])

  return agg


@functools.partial(
    pl.kernel,
    out_type=jax.ShapeDtypeStruct((NC, N_PAD, DEG_W), jnp.float32),
    mesh=_mesh,
    scratch_types=[
        pltpu.VMEM((EC, CHUNK), jnp.int32),
        pltpu.VMEM((CHUNK, DEG_W), jnp.float32),
        pltpu.VMEM_SHARED((N_PAD, DEG_W), jnp.float32),
    ],
    compiler_params=_sc_params,
)
def _deg_kernel(dst_hbm, ones_hbm, zeros_hbm, out_hbm,
                dst_v, ones_v, acc_sh):
  cid = lax.axis_index("c")
  sid = lax.axis_index("s")
  wid = sid * NC + cid
  pltpu.sync_copy(dst_hbm.at[wid], dst_v)
  pltpu.sync_copy(ones_hbm, ones_v)
  pltpu.sync_copy(zeros_hbm, acc_sh.at[pl.ds(sid * RPT, RPT)])
  plsc.subcore_barrier()

  def body(j, carry):
    pltpu.sync_copy(ones_v, acc_sh.at[dst_v.at[j]], add=True)
    return carry

  lax.fori_loop(0, EC, body, 0)
  plsc.subcore_barrier()
  pltpu.sync_copy(acc_sh.at[pl.ds(sid * RPT, RPT)],
                  out_hbm.at[cid, pl.ds(sid * RPT, RPT)])


def _tc_first(degA, degB, x, W):
  def body(da, db, x_ref, w_ref, xs_out, isd_out):
    deg = jnp.maximum(da[:, 0:1] + db[:, 0:1], 1.0)
    isd = lax.rsqrt(deg)
    t = jnp.dot(x_ref[...], w_ref[...], preferred_element_type=jnp.float32)
    xs_out[...] = t * isd
    isd_out[...] = jnp.broadcast_to(isd, (N, DEG_W))
  return pl.pallas_call(
      body,
      out_shape=[jax.ShapeDtypeStruct((N, W.shape[1]), jnp.float32),
                 jax.ShapeDtypeStruct((N, DEG_W), jnp.float32)],
  )(degA, degB, x, W)


def _tc_mid(aggA, aggB, isd, b, W, want_h):
  def body(aa, ab, isd_ref, b_ref, w_ref, *outs):
    isd = isd_ref[:, 0:1]
    h = jnp.maximum(isd * (aa[...] + ab[...]) + b_ref[...], 0.0)
    t = jnp.dot(h, w_ref[...], preferred_element_type=jnp.float32)
    outs[0][...] = t * isd
    if want_h:
      outs[1][...] = h
  out_shape = [jax.ShapeDtypeStruct((N, W.shape[1]), jnp.float32)]
  if want_h:
    out_shape.append(jax.ShapeDtypeStruct((N, b.shape[0]), jnp.float32))
  return pl.pallas_call(body, out_shape=out_shape)(
      aggA, aggB, isd, b.reshape(1, -1), W)


def _tc_scale(aggA, aggB, isd, b):
  def body(aa, ab, isd_ref, b_ref, out):
    isd = isd_ref[:, 0:1]
    out[...] = jnp.maximum(isd * (aa[...] + ab[...]) + b_ref[...], 0.0) * isd
  return pl.pallas_call(
      body, out_shape=jax.ShapeDtypeStruct((N, aggA.shape[1]), jnp.float32)
  )(aggA, aggB, isd, b.reshape(1, -1))


def _tc_last(aggA, aggB, isd, W, b):
  def body(aa, ab, isd_ref, w_ref, b_ref, out):
    isd = isd_ref[:, 0:1]
    t = jnp.dot(isd * (aa[...] + ab[...]), w_ref[...],
                preferred_element_type=jnp.float32)
    out[...] = t + b_ref[...]
  return pl.pallas_call(
      body, out_shape=jax.ShapeDtypeStruct((N, W.shape[1]), jnp.float32)
  )(aggA, aggB, isd, W, b.reshape(1, -1))


_agg64 = _make_agg(64)
_agg32 = _make_agg(32)


def _pad_rows(x):
  return jnp.pad(x, ((0, N_PAD - N), (0, 0)))


def kernel(edge_index, edge_weight, We1, be1, We2, be2, Wd1, bd1, Wd2, bd2):
  src = edge_index[0].astype(jnp.int32)
  dst = edge_index[1].astype(jnp.int32)
  fill = jnp.full((E_PAD - E,), N, jnp.int32)
  src_p = jnp.concatenate([src, fill]).reshape(NW, EC, CHUNK)
  dst_p = jnp.concatenate([dst, fill]).reshape(NW, EC, CHUNK)
  ones_rows = jnp.ones((CHUNK, DEG_W), jnp.float32)
  zeros8 = jnp.zeros((RPT, DEG_W), jnp.float32)

  deg2 = _deg_kernel(dst_p, ones_rows, zeros8)
  xs1, isd = _tc_first(deg2[0, :N], deg2[1, :N], edge_weight, We1)

  agg1 = _agg64(_pad_rows(xs1), src_p, dst_p)
  xs2 = _tc_mid(agg1[0, :N], agg1[1, :N], isd, be1, We2, want_h=False)[0]

  agg2 = _agg32(_pad_rows(xs2), src_p, dst_p)
  xs3, z = _tc_mid(agg2[0, :N], agg2[1, :N], isd, be2, Wd1, want_h=True)

  agg3 = _agg64(_pad_rows(xs3), src_p, dst_p)
  xs4 = _tc_scale(agg3[0, :N], agg3[1, :N], isd, bd1)

  agg4 = _agg64(_pad_rows(xs4), src_p, dst_p)
  recon = _tc_last(agg4[0, :N], agg4[1, :N], isd, Wd2, bd2)

  return (recon, z)


# speedup vs baseline: 2.7778x; 1.1012x over previous
import functools

import jax
import jax.numpy as jnp
from jax import lax
from jax.experimental import pallas as pl
from jax.experimental.pallas import tpu as pltpu
from jax.experimental.pallas import tpu_sc as plsc

N = 10000
E = 320000
NC = 2
NS = 16
NW = NC * NS
CHUNK = 128
EC = 81
KB = 3
E_PAD = NW * EC * CHUNK
N_PAD = 10240
RPT = N_PAD // NS
DEG_W = 8

_mesh = plsc.VectorSubcoreMesh(
    core_axis_name="c", subcore_axis_name="s", num_cores=NC, num_subcores=NS
)
_sc_params = pltpu.CompilerParams(use_tc_tiling_on_sc=False)


def _make_agg(D):
  R = EC // KB

  @functools.partial(
      pl.kernel,
      out_type=jax.ShapeDtypeStruct((NC, N_PAD, D), jnp.float32),
      mesh=_mesh,
      scratch_types=[
          pltpu.VMEM((EC, CHUNK), jnp.int32),
          pltpu.VMEM((EC, CHUNK), jnp.int32),
          pltpu.VMEM((KB * CHUNK, D), jnp.float32),
          pltpu.VMEM_SHARED((N_PAD, D), jnp.float32),
          pltpu.VMEM_SHARED((N_PAD, D), jnp.float32),
          pltpu.SemaphoreType.DMA((KB,)),
          pltpu.SemaphoreType.DMA((KB,)),
      ],
      compiler_params=_sc_params,
  )
  def agg(xs_hbm, src_hbm, dst_hbm, out_hbm,
          src_v, dst_v, rows_v, acc_sh, table_sh, gsem, ssem):
    cid = lax.axis_index("c")
    sid = lax.axis_index("s")
    wid = sid * NC + cid

    def rbuf(b):
      return rows_v.at[pl.ds(b * CHUNK, CHUNK)]

    def g_start(j, b):
      pltpu.async_copy(table_sh.at[src_v.at[j]], rbuf(b), gsem.at[b])

    def g_wait(j, b):
      pltpu.make_async_copy(table_sh.at[src_v.at[j]], rbuf(b),
                            gsem.at[b]).wait()

    def s_start(j, b):
      pltpu.async_copy(rbuf(b), acc_sh.at[dst_v.at[j]], ssem.at[b], add=True)

    def s_wait(j, b):
      pltpu.make_async_copy(rbuf(b), acc_sh.at[dst_v.at[j]],
                            ssem.at[b]).wait()

    stage = [
        pltpu.async_copy(src_hbm.at[wid], src_v, gsem.at[0]),
        pltpu.async_copy(dst_hbm.at[wid], dst_v, gsem.at[1]),
        pltpu.async_copy(xs_hbm.at[pl.ds(sid * RPT, RPT)],
                         table_sh.at[pl.ds(sid * RPT, RPT)], gsem.at[2]),
    ]
    zv = jnp.zeros((16,), jnp.float32)

    def zrow(i, carry):
      for k in range(D // 16):
        rows_v[i, pl.ds(k * 16, 16)] = zv
      return carry

    lax.fori_loop(0, KB * CHUNK, zrow, 0)
    off = 0
    zi = 0
    while off < RPT:
      n = min(KB * CHUNK, RPT - off)
      stage.append(
          pltpu.async_copy(rows_v.at[pl.ds(0, n)],
                           acc_sh.at[pl.ds(sid * RPT + off, n)], ssem.at[zi]))
      zi += 1
      off += n
    for d in stage:
      d.wait()
    plsc.subcore_barrier()

    g_start(0, 0)

    def round_body(r, carry):
      for b in range(KB):
        j = r * KB + b
        bn = (b + 1) % KB
        if b == KB - 1:
          @pl.when(r < R - 1)
          def _():
            s_wait(j + 1 - KB, bn)
            g_start(j + 1, bn)
        else:
          @pl.when(r > 0)
          def _():
            s_wait(j + 1 - KB, bn)
          g_start(j + 1, bn)
        g_wait(j, b)
        s_start(j, b)
      return carry

    lax.fori_loop(0, R, round_body, 0)
    for b in range(KB):
      s_wait(EC - KB + b, b)
    plsc.subcore_barrier()
    pltpu.sync_copy(acc_sh.at[pl.ds(sid * RPT, RPT)],
                    out_hbm.at[cid, pl.ds(sid * RPT, RPT)])

  return agg


@functools.partial(
    pl.kernel,
    out_type=jax.ShapeDtypeStruct((NC, N_PAD, DEG_W), jnp.float32),
    mesh=_mesh,
    scratch_types=[
        pltpu.VMEM((EC, CHUNK), jnp.int32),
        pltpu.VMEM((CHUNK, DEG_W), jnp.float32),
        pltpu.VMEM_SHARED((N_PAD, DEG_W), jnp.float32),
    ],
    compiler_params=_sc_params,
)
def _deg_kernel(dst_hbm, ones_hbm, zeros_hbm, out_hbm,
                dst_v, ones_v, acc_sh):
  cid = lax.axis_index("c")
  sid = lax.axis_index("s")
  wid = sid * NC + cid
  pltpu.sync_copy(dst_hbm.at[wid], dst_v)
  pltpu.sync_copy(ones_hbm, ones_v)
  pltpu.sync_copy(zeros_hbm, acc_sh.at[pl.ds(sid * RPT, RPT)])
  plsc.subcore_barrier()

  def body(j, carry):
    pltpu.sync_copy(ones_v, acc_sh.at[dst_v.at[j]], add=True)
    return carry

  lax.fori_loop(0, EC, body, 0)
  plsc.subcore_barrier()
  pltpu.sync_copy(acc_sh.at[pl.ds(sid * RPT, RPT)],
                  out_hbm.at[cid, pl.ds(sid * RPT, RPT)])


def _tc_first(deg2, x, W):
  def body(d2, x_ref, w_ref, xs_out, isd_out):
    deg = jnp.maximum(d2[0, :N, 0:1] + d2[1, :N, 0:1], 1.0)
    isd = lax.rsqrt(deg)
    t = jnp.dot(x_ref[...], w_ref[...], preferred_element_type=jnp.float32)
    xs_out[:N, :] = t * isd
    isd_out[...] = jnp.broadcast_to(isd, (N, DEG_W))
  return pl.pallas_call(
      body,
      out_shape=[jax.ShapeDtypeStruct((N_PAD, W.shape[1]), jnp.float32),
                 jax.ShapeDtypeStruct((N, DEG_W), jnp.float32)],
  )(deg2, x, W)


def _tc_mid(agg2, isd, b, W, want_h):
  def body(a2, isd_ref, b_ref, w_ref, *outs):
    isd = isd_ref[:, 0:1]
    h = jnp.maximum(isd * (a2[0, :N, :] + a2[1, :N, :]) + b_ref[...], 0.0)
    t = jnp.dot(h, w_ref[...], preferred_element_type=jnp.float32)
    outs[0][:N, :] = t * isd
    if want_h:
      outs[1][...] = h
  out_shape = [jax.ShapeDtypeStruct((N_PAD, W.shape[1]), jnp.float32)]
  if want_h:
    out_shape.append(jax.ShapeDtypeStruct((N, b.shape[0]), jnp.float32))
  return pl.pallas_call(body, out_shape=out_shape)(
      agg2, isd, b.reshape(1, -1), W)


def _tc_scale(agg2, isd, b):
  def body(a2, isd_ref, b_ref, out):
    isd = isd_ref[:, 0:1]
    out[:N, :] = jnp.maximum(
        isd * (a2[0, :N, :] + a2[1, :N, :]) + b_ref[...], 0.0) * isd
  return pl.pallas_call(
      body, out_shape=jax.ShapeDtypeStruct((N_PAD, agg2.shape[2]), jnp.float32)
  )(agg2, isd, b.reshape(1, -1))


def _tc_last(agg2, isd, W, b):
  def body(a2, isd_ref, w_ref, b_ref, out):
    isd = isd_ref[:, 0:1]
    t = jnp.dot(isd * (a2[0, :N, :] + a2[1, :N, :]), w_ref[...],
                preferred_element_type=jnp.float32)
    out[...] = t + b_ref[...]
  return pl.pallas_call(
      body, out_shape=jax.ShapeDtypeStruct((N, W.shape[1]), jnp.float32)
  )(agg2, isd, W, b.reshape(1, -1))


_agg64 = _make_agg(64)
_agg32 = _make_agg(32)


def kernel(edge_index, edge_weight, We1, be1, We2, be2, Wd1, bd1, Wd2, bd2):
  src = edge_index[0].astype(jnp.int32)
  dst = edge_index[1].astype(jnp.int32)
  fill = jnp.full((E_PAD - E,), N, jnp.int32)
  src_p = jnp.concatenate([src, fill]).reshape(NW, EC, CHUNK)
  dst_p = jnp.concatenate([dst, fill]).reshape(NW, EC, CHUNK)
  ones_rows = jnp.ones((CHUNK, DEG_W), jnp.float32)
  zeros8 = jnp.zeros((RPT, DEG_W), jnp.float32)

  deg2 = _deg_kernel(dst_p, ones_rows, zeros8)
  xs1, isd = _tc_first(deg2, edge_weight, We1)

  agg1 = _agg64(xs1, src_p, dst_p)
  xs2 = _tc_mid(agg1, isd, be1, We2, want_h=False)[0]

  agg2 = _agg32(xs2, src_p, dst_p)
  xs3, z = _tc_mid(agg2, isd, be2, Wd1, want_h=True)

  agg3 = _agg64(xs3, src_p, dst_p)
  xs4 = _tc_scale(agg3, isd, bd1)

  agg4 = _agg64(xs4, src_p, dst_p)
  recon = _tc_last(agg4, isd, Wd2, bd2)

  return (recon, z)


# speedup vs baseline: 2.7785x; 1.0003x over previous
import functools

import jax
import jax.numpy as jnp
from jax import lax
from jax.experimental import pallas as pl
from jax.experimental.pallas import tpu as pltpu
from jax.experimental.pallas import tpu_sc as plsc

N = 10000
E = 320000
NC = 2
NS = 16
NW = NC * NS
CHUNK = 128
EC = 81
KB = 3
E_PAD = NW * EC * CHUNK
N_PAD = 10240
RPT = N_PAD // NS
DEG_W = 8

_mesh = plsc.VectorSubcoreMesh(
    core_axis_name="c", subcore_axis_name="s", num_cores=NC, num_subcores=NS
)
_sc_params = pltpu.CompilerParams(
    use_tc_tiling_on_sc=False,
    disable_bounds_checks=True,
    disable_semaphore_checks=True,
)


def _make_agg(D):
  R = EC // KB

  @functools.partial(
      pl.kernel,
      out_type=jax.ShapeDtypeStruct((NC, N_PAD, D), jnp.float32),
      mesh=_mesh,
      scratch_types=[
          pltpu.VMEM((EC, CHUNK), jnp.int32),
          pltpu.VMEM((EC, CHUNK), jnp.int32),
          pltpu.VMEM((KB * CHUNK, D), jnp.float32),
          pltpu.VMEM_SHARED((N_PAD, D), jnp.float32),
          pltpu.VMEM_SHARED((N_PAD, D), jnp.float32),
          pltpu.SemaphoreType.DMA((KB,)),
          pltpu.SemaphoreType.DMA((KB,)),
      ],
      compiler_params=_sc_params,
  )
  def agg(xs_hbm, src_hbm, dst_hbm, out_hbm,
          src_v, dst_v, rows_v, acc_sh, table_sh, gsem, ssem):
    cid = lax.axis_index("c")
    sid = lax.axis_index("s")
    wid = sid * NC + cid

    def rbuf(b):
      return rows_v.at[pl.ds(b * CHUNK, CHUNK)]

    def g_start(j, b):
      pltpu.async_copy(table_sh.at[src_v.at[j]], rbuf(b), gsem.at[b])

    def g_wait(j, b):
      pltpu.make_async_copy(table_sh.at[src_v.at[j]], rbuf(b),
                            gsem.at[b]).wait()

    def s_start(j, b):
      pltpu.async_copy(rbuf(b), acc_sh.at[dst_v.at[j]], ssem.at[b], add=True)

    def s_wait(j, b):
      pltpu.make_async_copy(rbuf(b), acc_sh.at[dst_v.at[j]],
                            ssem.at[b]).wait()

    stage = [
        pltpu.async_copy(src_hbm.at[wid], src_v, gsem.at[0]),
        pltpu.async_copy(dst_hbm.at[wid], dst_v, gsem.at[1]),
        pltpu.async_copy(xs_hbm.at[pl.ds(sid * RPT, RPT)],
                         table_sh.at[pl.ds(sid * RPT, RPT)], gsem.at[2]),
    ]
    zv = jnp.zeros((16,), jnp.float32)

    def zrow(i, carry):
      for k in range(D // 16):
        rows_v[i, pl.ds(k * 16, 16)] = zv
      return carry

    lax.fori_loop(0, KB * CHUNK, zrow, 0)
    off = 0
    zi = 0
    while off < RPT:
      n = min(KB * CHUNK, RPT - off)
      stage.append(
          pltpu.async_copy(rows_v.at[pl.ds(0, n)],
                           acc_sh.at[pl.ds(sid * RPT + off, n)], ssem.at[zi]))
      zi += 1
      off += n
    for d in stage:
      d.wait()
    plsc.subcore_barrier()

    g_start(0, 0)

    def round_body(r, carry):
      for b in range(KB):
        j = r * KB + b
        bn = (b + 1) % KB
        if b == KB - 1:
          @pl.when(r < R - 1)
          def _():
            s_wait(j + 1 - KB, bn)
            g_start(j + 1, bn)
        else:
          @pl.when(r > 0)
          def _():
            s_wait(j + 1 - KB, bn)
          g_start(j + 1, bn)
        g_wait(j, b)
        s_start(j, b)
      return carry

    lax.fori_loop(0, R, round_body, 0)
    for b in range(KB):
      s_wait(EC - KB + b, b)
    plsc.subcore_barrier()
    pltpu.sync_copy(acc_sh.at[pl.ds(sid * RPT, RPT)],
                    out_hbm.at[cid, pl.ds(sid * RPT, RPT)])

  return agg


@functools.partial(
    pl.kernel,
    out_type=jax.ShapeDtypeStruct((NC, N_PAD, DEG_W), jnp.float32),
    mesh=_mesh,
    scratch_types=[
        pltpu.VMEM((EC, CHUNK), jnp.int32),
        pltpu.VMEM((CHUNK, DEG_W), jnp.float32),
        pltpu.VMEM_SHARED((N_PAD, DEG_W), jnp.float32),
    ],
    compiler_params=_sc_params,
)
def _deg_kernel(dst_hbm, ones_hbm, zeros_hbm, out_hbm,
                dst_v, ones_v, acc_sh):
  cid = lax.axis_index("c")
  sid = lax.axis_index("s")
  wid = sid * NC + cid
  pltpu.sync_copy(dst_hbm.at[wid], dst_v)
  pltpu.sync_copy(ones_hbm, ones_v)
  pltpu.sync_copy(zeros_hbm, acc_sh.at[pl.ds(sid * RPT, RPT)])
  plsc.subcore_barrier()

  def body(j, carry):
    pltpu.sync_copy(ones_v, acc_sh.at[dst_v.at[j]], add=True)
    return carry

  lax.fori_loop(0, EC, body, 0)
  plsc.subcore_barrier()
  pltpu.sync_copy(acc_sh.at[pl.ds(sid * RPT, RPT)],
                  out_hbm.at[cid, pl.ds(sid * RPT, RPT)])


def _tc_first(deg2, x, W):
  def body(d2, x_ref, w_ref, xs_out, isd_out):
    deg = jnp.maximum(d2[0, :N, 0:1] + d2[1, :N, 0:1], 1.0)
    isd = lax.rsqrt(deg)
    t = jnp.dot(x_ref[...], w_ref[...], preferred_element_type=jnp.float32)
    xs_out[:N, :] = t * isd
    isd_out[...] = jnp.broadcast_to(isd, (N, DEG_W))
  return pl.pallas_call(
      body,
      out_shape=[jax.ShapeDtypeStruct((N_PAD, W.shape[1]), jnp.float32),
                 jax.ShapeDtypeStruct((N, DEG_W), jnp.float32)],
  )(deg2, x, W)


def _tc_mid(agg2, isd, b, W, want_h):
  def body(a2, isd_ref, b_ref, w_ref, *outs):
    isd = isd_ref[:, 0:1]
    h = jnp.maximum(isd * (a2[0, :N, :] + a2[1, :N, :]) + b_ref[...], 0.0)
    t = jnp.dot(h, w_ref[...], preferred_element_type=jnp.float32)
    outs[0][:N, :] = t * isd
    if want_h:
      outs[1][...] = h
  out_shape = [jax.ShapeDtypeStruct((N_PAD, W.shape[1]), jnp.float32)]
  if want_h:
    out_shape.append(jax.ShapeDtypeStruct((N, b.shape[0]), jnp.float32))
  return pl.pallas_call(body, out_shape=out_shape)(
      agg2, isd, b.reshape(1, -1), W)


def _tc_scale(agg2, isd, b):
  def body(a2, isd_ref, b_ref, out):
    isd = isd_ref[:, 0:1]
    out[:N, :] = jnp.maximum(
        isd * (a2[0, :N, :] + a2[1, :N, :]) + b_ref[...], 0.0) * isd
  return pl.pallas_call(
      body, out_shape=jax.ShapeDtypeStruct((N_PAD, agg2.shape[2]), jnp.float32)
  )(agg2, isd, b.reshape(1, -1))


def _tc_last(agg2, isd, W, b):
  def body(a2, isd_ref, w_ref, b_ref, out):
    isd = isd_ref[:, 0:1]
    t = jnp.dot(isd * (a2[0, :N, :] + a2[1, :N, :]), w_ref[...],
                preferred_element_type=jnp.float32)
    out[...] = t + b_ref[...]
  return pl.pallas_call(
      body, out_shape=jax.ShapeDtypeStruct((N, W.shape[1]), jnp.float32)
  )(agg2, isd, W, b.reshape(1, -1))


_agg64 = _make_agg(64)
_agg32 = _make_agg(32)


def kernel(edge_index, edge_weight, We1, be1, We2, be2, Wd1, bd1, Wd2, bd2):
  src = edge_index[0].astype(jnp.int32)
  dst = edge_index[1].astype(jnp.int32)
  fill = jnp.full((E_PAD - E,), N, jnp.int32)
  src_p = jnp.concatenate([src, fill]).reshape(NW, EC, CHUNK)
  dst_p = jnp.concatenate([dst, fill]).reshape(NW, EC, CHUNK)
  ones_rows = jnp.ones((CHUNK, DEG_W), jnp.float32)
  zeros8 = jnp.zeros((RPT, DEG_W), jnp.float32)

  deg2 = _deg_kernel(dst_p, ones_rows, zeros8)
  xs1, isd = _tc_first(deg2, edge_weight, We1)

  agg1 = _agg64(xs1, src_p, dst_p)
  xs2 = _tc_mid(agg1, isd, be1, We2, want_h=False)[0]

  agg2 = _agg32(xs2, src_p, dst_p)
  xs3, z = _tc_mid(agg2, isd, be2, Wd1, want_h=True)

  agg3 = _agg64(xs3, src_p, dst_p)
  xs4 = _tc_scale(agg3, isd, bd1)

  agg4 = _agg64(xs4, src_p, dst_p)
  recon = _tc_last(agg4, isd, Wd2, bd2)

  return (recon, z)
